# Initial kernel scaffold; baseline (speedup 1.0000x reference)
import jax
import jax.numpy as jnp
from jax.experimental import pallas as pl


def kernel(sysevr_input, x, edge_index, batch, Ws, bs, Wg0, bg0, Wg1, bg1, Wg2, bg2, W1, b1, W2, b2):
    raise NotImplementedError("write your pallas kernel here")


# speedup vs baseline: 5.7369x; 5.7369x over previous
import functools

import jax
import jax.numpy as jnp
from jax import lax
from jax.experimental import pallas as pl
from jax.experimental.pallas import tpu as pltpu
from jax.experimental.pallas import tpu_sc as plsc

N = 10000
D = 128
B = 32
DSYS = 100
SYS_OUT = 512

NCORES = 2
NSUB = 16
NW = NCORES * NSUB
N_PAD = 10240
ROWS_PER_TILE = N_PAD // NSUB
CHUNK = 40
SEG = 32
NSEG = 8
NCH = SEG * NSEG
EPT = NCH * CHUNK
E_PAD = NW * EPT
DEGW = 128
BR = 1280
NB = N_PAD // BR

_mesh = plsc.VectorSubcoreMesh(
    core_axis_name="c", subcore_axis_name="s",
    num_cores=NCORES, num_subcores=NSUB,
)


@functools.partial(
    pl.kernel,
    out_type=jax.ShapeDtypeStruct((NCORES, N_PAD, DEGW), jnp.float32),
    mesh=_mesh,
    scratch_types=[
        pltpu.VMEM((SEG, CHUNK), jnp.int32),
        pltpu.VMEM((CHUNK, DEGW), jnp.float32),
        pltpu.VMEM_SHARED((N_PAD, DEGW), jnp.float32),
    ],
)
def _sc_deg(dst_hbm, ones_hbm, zeros_hbm, out_hbm, seg_v, ones_v, acc):
    c = lax.axis_index("c")
    s = lax.axis_index("s")
    wid = c * NSUB + s
    pltpu.sync_copy(ones_hbm, ones_v)
    pltpu.sync_copy(zeros_hbm, acc.at[pl.ds(s * ROWS_PER_TILE, ROWS_PER_TILE)])
    plsc.subcore_barrier()

    def seg_step(g, carry):
        pltpu.sync_copy(dst_hbm.at[wid, g], seg_v)

        def step(k, carry2):
            pltpu.sync_copy(ones_v, acc.at[seg_v.at[k]], add=True)
            return carry2

        lax.fori_loop(0, SEG, step, 0)
        return carry

    lax.fori_loop(0, NSEG, seg_step, 0)
    plsc.subcore_barrier()
    pltpu.sync_copy(
        acc.at[pl.ds(s * ROWS_PER_TILE, ROWS_PER_TILE)],
        out_hbm.at[c, pl.ds(s * ROWS_PER_TILE, ROWS_PER_TILE)],
    )


@functools.partial(
    pl.kernel,
    out_type=jax.ShapeDtypeStruct((NCORES, N_PAD, D), jnp.float32),
    mesh=_mesh,
    scratch_types=[
        pltpu.VMEM((SEG, CHUNK), jnp.int32),
        pltpu.VMEM((SEG, CHUNK), jnp.int32),
        pltpu.VMEM((CHUNK, D), jnp.float32),
        pltpu.VMEM((CHUNK, D), jnp.float32),
        pltpu.VMEM_SHARED((N_PAD, D), jnp.float32),
        pltpu.SemaphoreType.DMA,
        pltpu.SemaphoreType.DMA,
    ],
)
def _sc_conv(h_hbm, src_hbm, dst_hbm, zeros_hbm, out_hbm,
             src_v, dst_v, rows0, rows1, acc, sem0, sem1):
    c = lax.axis_index("c")
    s = lax.axis_index("s")
    wid = c * NSUB + s
    pltpu.sync_copy(zeros_hbm, acc.at[pl.ds(s * ROWS_PER_TILE, ROWS_PER_TILE)])
    plsc.subcore_barrier()

    def seg_step(g, carry):
        pltpu.sync_copy(src_hbm.at[wid, g], src_v)
        pltpu.sync_copy(dst_hbm.at[wid, g], dst_v)
        pltpu.async_copy(h_hbm.at[src_v.at[0]], rows0, sem0)

        def step(i, carry2):
            j0 = 2 * i
            j1 = 2 * i + 1
            pltpu.async_copy(h_hbm.at[src_v.at[j1]], rows1, sem1)
            pltpu.make_async_copy(h_hbm.at[src_v.at[j0]], rows0, sem0).wait()
            pltpu.sync_copy(rows0, acc.at[dst_v.at[j0]], add=True)

            @pl.when(j0 + 2 < SEG)
            def _():
                pltpu.async_copy(h_hbm.at[src_v.at[j0 + 2]], rows0, sem0)

            pltpu.make_async_copy(h_hbm.at[src_v.at[j1]], rows1, sem1).wait()
            pltpu.sync_copy(rows1, acc.at[dst_v.at[j1]], add=True)
            return carry2

        lax.fori_loop(0, SEG // 2, step, 0)
        return carry

    lax.fori_loop(0, NSEG, seg_step, 0)
    plsc.subcore_barrier()
    pltpu.sync_copy(
        acc.at[pl.ds(s * ROWS_PER_TILE, ROWS_PER_TILE)],
        out_hbm.at[c, pl.ds(s * ROWS_PER_TILE, ROWS_PER_TILE)],
    )


def _tc_prep(deg_parts, x_pad):
    def body(deg_ref, x_ref, dis_ref, h1_ref):
        deg = deg_ref[0, :, 0:1] + deg_ref[1, :, 0:1]
        dis = lax.rsqrt(jnp.clip(deg, 1.0, None))
        dis = jnp.broadcast_to(dis, (BR, D))
        r = pl.program_id(0)
        rows = lax.broadcasted_iota(jnp.int32, (BR, D), 0) + r * BR
        dis = jnp.where(rows < N, dis, 0.0)
        dis_ref[...] = dis
        h1_ref[...] = x_ref[...] * dis

    return pl.pallas_call(
        body,
        grid=(NB,),
        in_specs=[
            pl.BlockSpec((NCORES, BR, DEGW), lambda r: (0, r, 0)),
            pl.BlockSpec((BR, D), lambda r: (r, 0)),
        ],
        out_specs=[pl.BlockSpec((BR, D), lambda r: (r, 0))] * 2,
        out_shape=[jax.ShapeDtypeStruct((N_PAD, D), jnp.float32)] * 2,
    )(deg_parts, x_pad)


def _tc_layer(parts, dis_m, W, b):
    def body(p_ref, dis_ref, w_ref, b_ref, o_ref):
        dis = dis_ref[...]
        z = (p_ref[0] + p_ref[1]) * dis
        h = jnp.dot(z, w_ref[...], preferred_element_type=jnp.float32) + b_ref[...]
        o_ref[...] = jnp.maximum(h, 0.0) * dis

    return pl.pallas_call(
        body,
        grid=(NB,),
        in_specs=[
            pl.BlockSpec((NCORES, BR, D), lambda r: (0, r, 0)),
            pl.BlockSpec((BR, D), lambda r: (r, 0)),
            pl.BlockSpec((D, D), lambda r: (0, 0)),
            pl.BlockSpec((1, D), lambda r: (0, 0)),
        ],
        out_specs=pl.BlockSpec((BR, D), lambda r: (r, 0)),
        out_shape=jax.ShapeDtypeStruct((N_PAD, D), jnp.float32),
    )(parts, dis_m, W, b)


def _tc_final(parts, dis_m, batch_col, Wg2, bg2, si, Ws, bs,
              W1s, W1g, b1, W2, b2):
    def body(p_ref, dis_ref, bt_ref, wg_ref, bg_ref, si_ref, ws_ref, bs_ref,
             w1s_ref, w1g_ref, b1_ref, w2_ref, b2_ref, out_ref,
             sums_acc, cnt_acc):
        r = pl.program_id(0)

        @pl.when(r == 0)
        def _():
            sums_acc[...] = jnp.zeros((B, D), jnp.float32)
            cnt_acc[...] = jnp.zeros((B, D), jnp.float32)

        z = (p_ref[0] + p_ref[1]) * dis_ref[...]
        post = jnp.dot(z, wg_ref[...], preferred_element_type=jnp.float32)
        post = post + bg_ref[...]
        gids = lax.broadcasted_iota(jnp.int32, (BR, B), 1)
        oh = (bt_ref[...] == gids).astype(jnp.float32)
        dn = (((0,), (0,)), ((), ()))
        sums_acc[...] += lax.dot_general(
            oh, post, dn, preferred_element_type=jnp.float32)
        cnt_acc[...] += lax.dot_general(
            oh, jnp.ones((BR, D), jnp.float32), dn,
            preferred_element_type=jnp.float32)

        @pl.when(r == NB - 1)
        def _():
            ivdet = sums_acc[...] / jnp.clip(cnt_acc[...], 1.0, None)
            sys = jnp.dot(si_ref[...], ws_ref[...],
                          preferred_element_type=jnp.float32) + bs_ref[...]
            sys = jnp.maximum(sys, 0.0)
            hh = (jnp.dot(sys, w1s_ref[...], preferred_element_type=jnp.float32)
                  + jnp.dot(ivdet, w1g_ref[...],
                            preferred_element_type=jnp.float32)
                  + b1_ref[...])
            hh = jnp.maximum(hh, 0.0)
            out_ref[...] = jnp.dot(
                hh, w2_ref[...], preferred_element_type=jnp.float32) + b2_ref[...]

    full = lambda shape: pl.BlockSpec(shape, lambda r: tuple(0 for _ in shape))
    return pl.pallas_call(
        body,
        grid=(NB,),
        in_specs=[
            pl.BlockSpec((NCORES, BR, D), lambda r: (0, r, 0)),
            pl.BlockSpec((BR, D), lambda r: (r, 0)),
            pl.BlockSpec((BR, 1), lambda r: (r, 0)),
            full((D, D)),
            full((1, D)),
            full((B, DSYS)),
            full((DSYS, SYS_OUT)),
            full((1, SYS_OUT)),
            full((SYS_OUT, 128)),
            full((D, 128)),
            full((1, 128)),
            full((128, 2)),
            full((1, 2)),
        ],
        out_specs=pl.BlockSpec((B, 2), lambda r: (0, 0)),
        out_shape=jax.ShapeDtypeStruct((B, 2), jnp.float32),
        scratch_shapes=[
            pltpu.VMEM((B, D), jnp.float32),
            pltpu.VMEM((B, D), jnp.float32),
        ],
    )(parts, dis_m, batch_col, Wg2, bg2, si, Ws, bs, W1s, W1g, b1, W2, b2)


def kernel(sysevr_input, x, edge_index, batch, Ws, bs,
           Wg0, bg0, Wg1, bg1, Wg2, bg2, W1, b1, W2, b2):
    E = edge_index.shape[1]
    pad_e = E_PAD - E
    src = edge_index[0].astype(jnp.int32)
    dst = edge_index[1].astype(jnp.int32)
    src_r = jnp.concatenate(
        [src, jnp.full((pad_e,), N, jnp.int32)]).reshape(NW, NSEG, SEG, CHUNK)
    dst_r = jnp.concatenate(
        [dst, jnp.full((pad_e,), N, jnp.int32)]).reshape(NW, NSEG, SEG, CHUNK)
    x_pad = jnp.pad(x, ((0, N_PAD - N), (0, 0)))
    zeros_rows = jnp.zeros((ROWS_PER_TILE, D), jnp.float32)
    zeros_deg = jnp.zeros((ROWS_PER_TILE, DEGW), jnp.float32)
    ones_deg = jnp.ones((CHUNK, DEGW), jnp.float32)
    batch_col = jnp.concatenate(
        [batch.astype(jnp.int32), jnp.full((N_PAD - N,), B + 7, jnp.int32)]
    ).reshape(N_PAD, 1)

    deg_parts = _sc_deg(dst_r, ones_deg, zeros_deg)
    dis_m, h = _tc_prep(deg_parts, x_pad)
    parts = _sc_conv(h, src_r, dst_r, zeros_rows)
    h = _tc_layer(parts, dis_m, Wg0, bg0.reshape(1, D))
    parts = _sc_conv(h, src_r, dst_r, zeros_rows)
    h = _tc_layer(parts, dis_m, Wg1, bg1.reshape(1, D))
    parts = _sc_conv(h, src_r, dst_r, zeros_rows)
    out = _tc_final(parts, dis_m, batch_col, Wg2, bg2.reshape(1, D),
                    sysevr_input, Ws, bs.reshape(1, SYS_OUT),
                    W1[:SYS_OUT], W1[SYS_OUT:], b1.reshape(1, 128),
                    W2, b2.reshape(1, 2))
    return out


# speedup vs baseline: 9.8121x; 1.7103x over previous
import functools

import jax
import jax.numpy as jnp
from jax import lax
from jax.experimental import pallas as pl
from jax.experimental.pallas import tpu as pltpu
from jax.experimental.pallas import tpu_sc as plsc

N = 10000
D = 128
B = 32
DSYS = 100
SYS_OUT = 512

NCORES = 2
NSUB = 16
N_PAD = 10112
ROWS_PER_TILE = N_PAD // NSUB
CHUNK = 40
NCHT = 8000
SEG = 8
BR = 1264
NB = N_PAD // BR

_mesh = plsc.VectorSubcoreMesh(
    core_axis_name="c", subcore_axis_name="s",
    num_cores=NCORES, num_subcores=NSUB,
)


def _conv_partition(c, s):
    half = s < 8
    nch = jnp.where(c == 0, jnp.where(half, 368, 352),
                    jnp.where(half, 144, 136))
    base0 = jnp.where(half, s * 368, 8 * 368 + (s - 8) * 352)
    base1 = 5760 + jnp.where(half, s * 144, 8 * 144 + (s - 8) * 136)
    cbase = pl.multiple_of(jnp.where(c == 0, base0, base1), 8)
    return nch, cbase


def _deg_partition(wid):
    nch = jnp.where(wid < 8, 256, 248)
    cbase = pl.multiple_of(
        jnp.where(wid < 8, wid * 256, 2048 + (wid - 8) * 248), 8)
    return nch, cbase


@functools.partial(
    pl.kernel,
    out_type=jax.ShapeDtypeStruct((NCORES, N_PAD, D), jnp.float32),
    mesh=_mesh,
    scratch_types=[
        pltpu.VMEM((SEG, CHUNK), jnp.int32),
        pltpu.VMEM((CHUNK, D), jnp.float32),
        pltpu.VMEM_SHARED((N_PAD, D), jnp.float32),
    ],
)
def _sc_deg(dst_hbm, ones_hbm, zeros_hbm, out_hbm, seg_v, ones_v, acc):
    c = lax.axis_index("c")
    s = lax.axis_index("s")
    wid = c * NSUB + s
    nch, cbase = _deg_partition(wid)
    pltpu.sync_copy(ones_hbm, ones_v)
    pltpu.sync_copy(zeros_hbm, acc.at[pl.ds(pl.multiple_of(s * ROWS_PER_TILE, 8), ROWS_PER_TILE)])
    plsc.subcore_barrier()

    def seg_step(g, carry):
        pltpu.sync_copy(dst_hbm.at[pl.ds(pl.multiple_of(cbase + g * SEG, 8), SEG)], seg_v)

        def step(k, carry2):
            pltpu.sync_copy(ones_v, acc.at[seg_v.at[k]], add=True)
            return carry2

        lax.fori_loop(0, SEG, step, 0)
        return carry

    lax.fori_loop(0, lax.div(nch, SEG), seg_step, 0)
    plsc.subcore_barrier()
    pltpu.sync_copy(
        acc.at[pl.ds(pl.multiple_of(s * ROWS_PER_TILE, 8), ROWS_PER_TILE)],
        out_hbm.at[c, pl.ds(pl.multiple_of(s * ROWS_PER_TILE, 8), ROWS_PER_TILE)],
    )


@functools.partial(
    pl.kernel,
    out_type=jax.ShapeDtypeStruct((NCORES, N_PAD, D), jnp.float32),
    mesh=_mesh,
    scratch_types=[
        pltpu.VMEM((2, SEG, CHUNK), jnp.int32),
        pltpu.VMEM((2, SEG, CHUNK), jnp.int32),
        pltpu.VMEM((CHUNK, D), jnp.float32),
        pltpu.VMEM((CHUNK, D), jnp.float32),
        pltpu.VMEM_SHARED((N_PAD, D), jnp.float32),
        pltpu.SemaphoreType.DMA,
        pltpu.SemaphoreType.DMA,
    ],
)
def _sc_conv(h_hbm, srcc_hbm, dstc_hbm, zeros_hbm, out_hbm,
             srcseg, dstseg, rows0, rows1, acc, sem0, sem1):
    c = lax.axis_index("c")
    s = lax.axis_index("s")
    nch, cbase = _conv_partition(c, s)
    pltpu.sync_copy(zeros_hbm, acc.at[pl.ds(pl.multiple_of(s * ROWS_PER_TILE, 8), ROWS_PER_TILE)])
    plsc.subcore_barrier()
    pltpu.sync_copy(srcc_hbm.at[pl.ds(cbase, SEG)], srcseg.at[0])
    pltpu.sync_copy(dstc_hbm.at[pl.ds(cbase, SEG)], dstseg.at[0])
    pltpu.async_copy(h_hbm.at[srcseg.at[0, 0]], rows0, sem0)

    def pair(i, carry):
        j0 = 2 * i
        q0 = lax.rem(lax.div(j0, SEG), 2)
        k0 = lax.rem(j0, SEG)
        pltpu.async_copy(h_hbm.at[srcseg.at[q0, k0 + 1]], rows1, sem1)
        pltpu.make_async_copy(h_hbm.at[srcseg.at[q0, k0]], rows0, sem0).wait()
        pltpu.sync_copy(rows0, acc.at[dstseg.at[q0, k0]], add=True)
        j2 = j0 + 2
        q2 = lax.rem(lax.div(j2, SEG), 2)
        k2 = lax.rem(j2, SEG)

        @pl.when(jnp.logical_and(k0 == SEG - 2, j2 < nch))
        def _():
            pltpu.sync_copy(srcc_hbm.at[pl.ds(pl.multiple_of(cbase + j2, 8), SEG)], srcseg.at[q2])
            pltpu.sync_copy(dstc_hbm.at[pl.ds(pl.multiple_of(cbase + j2, 8), SEG)], dstseg.at[q2])

        @pl.when(j2 < nch)
        def _():
            pltpu.async_copy(h_hbm.at[srcseg.at[q2, k2]], rows0, sem0)

        pltpu.make_async_copy(h_hbm.at[srcseg.at[q0, k0 + 1]], rows1, sem1).wait()
        pltpu.sync_copy(rows1, acc.at[dstseg.at[q0, k0 + 1]], add=True)
        return carry

    lax.fori_loop(0, lax.div(nch, 2), pair, 0)
    plsc.subcore_barrier()
    pltpu.sync_copy(
        acc.at[pl.ds(pl.multiple_of(s * ROWS_PER_TILE, 8), ROWS_PER_TILE)],
        out_hbm.at[c, pl.ds(pl.multiple_of(s * ROWS_PER_TILE, 8), ROWS_PER_TILE)],
    )


def _tc_prep(deg_parts, x):
    def body(deg_ref, x_ref, dis_ref, h1_ref):
        deg = deg_ref[0] + deg_ref[1]
        dis = lax.rsqrt(jnp.clip(deg, 1.0, None))
        dis_ref[...] = dis
        h1_ref[...] = x_ref[...] * dis

    return pl.pallas_call(
        body,
        grid=(NB,),
        in_specs=[
            pl.BlockSpec((NCORES, BR, D), lambda r: (0, r, 0)),
            pl.BlockSpec((BR, D), lambda r: (r, 0)),
        ],
        out_specs=[pl.BlockSpec((BR, D), lambda r: (r, 0))] * 2,
        out_shape=[jax.ShapeDtypeStruct((N_PAD, D), jnp.float32)] * 2,
    )(deg_parts, x)


def _tc_layer(parts, dis_m, W, b):
    def body(p_ref, dis_ref, w_ref, b_ref, o_ref):
        dis = dis_ref[...]
        z = (p_ref[0] + p_ref[1]) * dis
        h = jnp.dot(z, w_ref[...], preferred_element_type=jnp.float32) + b_ref[...]
        o_ref[...] = jnp.maximum(h, 0.0) * dis

    return pl.pallas_call(
        body,
        grid=(NB,),
        in_specs=[
            pl.BlockSpec((NCORES, BR, D), lambda r: (0, r, 0)),
            pl.BlockSpec((BR, D), lambda r: (r, 0)),
            pl.BlockSpec((D, D), lambda r: (0, 0)),
            pl.BlockSpec((1, D), lambda r: (0, 0)),
        ],
        out_specs=pl.BlockSpec((BR, D), lambda r: (r, 0)),
        out_shape=jax.ShapeDtypeStruct((N_PAD, D), jnp.float32),
    )(parts, dis_m, W, b)


def _tc_final(parts, dis_m, batch_col, Wg2, bg2, si, Ws, bs,
              W1s, W1g, b1, W2, b2):
    def body(p_ref, dis_ref, bt_ref, wg_ref, bg_ref, si_ref, ws_ref, bs_ref,
             w1s_ref, w1g_ref, b1_ref, w2_ref, b2_ref, out_ref,
             sums_acc, cnt_acc):
        r = pl.program_id(0)

        @pl.when(r == 0)
        def _():
            sums_acc[...] = jnp.zeros((B, D), jnp.float32)
            cnt_acc[...] = jnp.zeros((B, D), jnp.float32)

        z = (p_ref[0] + p_ref[1]) * dis_ref[...]
        post = jnp.dot(z, wg_ref[...], preferred_element_type=jnp.float32)
        post = post + bg_ref[...]
        gids = lax.broadcasted_iota(jnp.int32, (BR, B), 1)
        oh = (bt_ref[...] == gids).astype(jnp.float32)
        dn = (((0,), (0,)), ((), ()))
        sums_acc[...] += lax.dot_general(
            oh, post, dn, preferred_element_type=jnp.float32)
        cnt_acc[...] += lax.dot_general(
            oh, jnp.ones((BR, D), jnp.float32), dn,
            preferred_element_type=jnp.float32)

        @pl.when(r == NB - 1)
        def _():
            ivdet = sums_acc[...] / jnp.clip(cnt_acc[...], 1.0, None)
            sys = jnp.dot(si_ref[...], ws_ref[...],
                          preferred_element_type=jnp.float32) + bs_ref[...]
            sys = jnp.maximum(sys, 0.0)
            hh = (jnp.dot(sys, w1s_ref[...], preferred_element_type=jnp.float32)
                  + jnp.dot(ivdet, w1g_ref[...],
                            preferred_element_type=jnp.float32)
                  + b1_ref[...])
            hh = jnp.maximum(hh, 0.0)
            out_ref[...] = jnp.dot(
                hh, w2_ref[...], preferred_element_type=jnp.float32) + b2_ref[...]

    full = lambda shape: pl.BlockSpec(shape, lambda r: tuple(0 for _ in shape))
    return pl.pallas_call(
        body,
        grid=(NB,),
        in_specs=[
            pl.BlockSpec((NCORES, BR, D), lambda r: (0, r, 0)),
            pl.BlockSpec((BR, D), lambda r: (r, 0)),
            pl.BlockSpec((BR, 1), lambda r: (r, 0)),
            full((D, D)),
            full((1, D)),
            full((B, DSYS)),
            full((DSYS, SYS_OUT)),
            full((1, SYS_OUT)),
            full((SYS_OUT, 128)),
            full((D, 128)),
            full((1, 128)),
            full((128, 2)),
            full((1, 2)),
        ],
        out_specs=pl.BlockSpec((B, 2), lambda r: (0, 0)),
        out_shape=jax.ShapeDtypeStruct((B, 2), jnp.float32),
        scratch_shapes=[
            pltpu.VMEM((B, D), jnp.float32),
            pltpu.VMEM((B, D), jnp.float32),
        ],
    )(parts, dis_m, batch_col, Wg2, bg2, si, Ws, bs, W1s, W1g, b1, W2, b2)


def kernel(sysevr_input, x, edge_index, batch, Ws, bs,
           Wg0, bg0, Wg1, bg1, Wg2, bg2, W1, b1, W2, b2):
    src_ch = edge_index[0].astype(jnp.int32).reshape(NCHT, CHUNK)
    dst_ch = edge_index[1].astype(jnp.int32).reshape(NCHT, CHUNK)
    x_pad = jnp.pad(x, ((0, N_PAD - N), (0, 0)))
    zeros_rows = jnp.zeros((ROWS_PER_TILE, D), jnp.float32)
    ones_rows = jnp.ones((CHUNK, D), jnp.float32)
    batch_col = jnp.concatenate(
        [batch.astype(jnp.int32), jnp.full((N_PAD - N,), B + 7, jnp.int32)]
    ).reshape(N_PAD, 1)

    deg_parts = _sc_deg(dst_ch, ones_rows, zeros_rows)
    dis_m, h = _tc_prep(deg_parts, x_pad)
    parts = _sc_conv(h, src_ch, dst_ch, zeros_rows)
    h = _tc_layer(parts, dis_m, Wg0, bg0.reshape(1, D))
    parts = _sc_conv(h, src_ch, dst_ch, zeros_rows)
    h = _tc_layer(parts, dis_m, Wg1, bg1.reshape(1, D))
    parts = _sc_conv(h, src_ch, dst_ch, zeros_rows)
    out = _tc_final(parts, dis_m, batch_col, Wg2, bg2.reshape(1, D),
                    sysevr_input, Ws, bs.reshape(1, SYS_OUT),
                    W1[:SYS_OUT], W1[SYS_OUT:], b1.reshape(1, 128),
                    W2, b2.reshape(1, 2))
    return out


# speedup vs baseline: 13.0047x; 1.3254x over previous
import functools

import jax
import jax.numpy as jnp
from jax import lax
from jax.experimental import pallas as pl
from jax.experimental.pallas import tpu as pltpu
from jax.experimental.pallas import tpu_sc as plsc

N = 10000
D = 128
B = 32
DSYS = 100
SYS_OUT = 512

NCORES = 2
NSUB = 16
N_PAD = 10112
ROWS_PER_TILE = N_PAD // NSUB
CHUNK = 40
NCHT = 8000
SEG = 8
BR = 1264
NB = N_PAD // BR

_mesh = plsc.VectorSubcoreMesh(
    core_axis_name="c", subcore_axis_name="s",
    num_cores=NCORES, num_subcores=NSUB,
)


def _conv_partition(c, s):
    half = s < 12
    nch = jnp.where(half, 248, 256)
    base_in_core = jnp.where(half, s * 248, 12 * 248 + (s - 12) * 256)
    cbase = pl.multiple_of(c * 4000 + base_in_core, 8)
    return nch, cbase


def _deg_partition(wid):
    nch = jnp.where(wid < 8, 256, 248)
    cbase = pl.multiple_of(
        jnp.where(wid < 8, wid * 256, 2048 + (wid - 8) * 248), 8)
    return nch, cbase


@functools.partial(
    pl.kernel,
    out_type=jax.ShapeDtypeStruct((NCORES, N_PAD, D), jnp.float32),
    mesh=_mesh,
    scratch_types=[
        pltpu.VMEM((SEG, CHUNK), jnp.int32),
        pltpu.VMEM((CHUNK, D), jnp.float32),
        pltpu.VMEM_SHARED((N_PAD, D), jnp.float32),
    ],
)
def _sc_deg(dst_hbm, ones_hbm, zeros_hbm, out_hbm, seg_v, ones_v, acc):
    c = lax.axis_index("c")
    s = lax.axis_index("s")
    wid = c * NSUB + s
    nch, cbase = _deg_partition(wid)
    pltpu.sync_copy(ones_hbm, ones_v)
    pltpu.sync_copy(zeros_hbm, acc.at[pl.ds(pl.multiple_of(s * ROWS_PER_TILE, 8), ROWS_PER_TILE)])
    plsc.subcore_barrier()

    def seg_step(g, carry):
        pltpu.sync_copy(dst_hbm.at[pl.ds(pl.multiple_of(cbase + g * SEG, 8), SEG)], seg_v)

        def step(k, carry2):
            pltpu.sync_copy(ones_v, acc.at[seg_v.at[k]], add=True)
            return carry2

        lax.fori_loop(0, SEG, step, 0)
        return carry

    lax.fori_loop(0, lax.div(nch, SEG), seg_step, 0)
    plsc.subcore_barrier()
    pltpu.sync_copy(
        acc.at[pl.ds(pl.multiple_of(s * ROWS_PER_TILE, 8), ROWS_PER_TILE)],
        out_hbm.at[c, pl.ds(pl.multiple_of(s * ROWS_PER_TILE, 8), ROWS_PER_TILE)],
    )


@functools.partial(
    pl.kernel,
    out_type=jax.ShapeDtypeStruct((NCORES, N_PAD, D), jnp.float32),
    mesh=_mesh,
    scratch_types=[
        pltpu.VMEM((2, SEG, CHUNK), jnp.int32),
        pltpu.VMEM((2, SEG, CHUNK), jnp.int32),
        pltpu.VMEM((CHUNK, D), jnp.float32),
        pltpu.VMEM((CHUNK, D), jnp.float32),
        pltpu.VMEM_SHARED((N_PAD, D), jnp.float32),
        pltpu.SemaphoreType.DMA,
        pltpu.SemaphoreType.DMA,
    ],
)
def _sc_conv(h_hbm, srcc_hbm, dstc_hbm, zeros_hbm, out_hbm,
             srcseg, dstseg, rows0, rows1, acc, sem0, sem1):
    c = lax.axis_index("c")
    s = lax.axis_index("s")
    nch, cbase = _conv_partition(c, s)
    pltpu.sync_copy(zeros_hbm, acc.at[pl.ds(pl.multiple_of(s * ROWS_PER_TILE, 8), ROWS_PER_TILE)])
    plsc.subcore_barrier()
    pltpu.sync_copy(srcc_hbm.at[pl.ds(cbase, SEG)], srcseg.at[0])
    pltpu.sync_copy(dstc_hbm.at[pl.ds(cbase, SEG)], dstseg.at[0])
    pltpu.async_copy(h_hbm.at[srcseg.at[0, 0]], rows0, sem0)

    def pair(i, carry):
        j0 = 2 * i
        q0 = lax.rem(lax.div(j0, SEG), 2)
        k0 = lax.rem(j0, SEG)
        pltpu.async_copy(h_hbm.at[srcseg.at[q0, k0 + 1]], rows1, sem1)
        pltpu.make_async_copy(h_hbm.at[srcseg.at[q0, k0]], rows0, sem0).wait()
        pltpu.sync_copy(rows0, acc.at[dstseg.at[q0, k0]], add=True)
        j2 = j0 + 2
        q2 = lax.rem(lax.div(j2, SEG), 2)
        k2 = lax.rem(j2, SEG)

        @pl.when(jnp.logical_and(k0 == SEG - 2, j2 < nch))
        def _():
            pltpu.sync_copy(srcc_hbm.at[pl.ds(pl.multiple_of(cbase + j2, 8), SEG)], srcseg.at[q2])
            pltpu.sync_copy(dstc_hbm.at[pl.ds(pl.multiple_of(cbase + j2, 8), SEG)], dstseg.at[q2])

        @pl.when(j2 < nch)
        def _():
            pltpu.async_copy(h_hbm.at[srcseg.at[q2, k2]], rows0, sem0)

        pltpu.make_async_copy(h_hbm.at[srcseg.at[q0, k0 + 1]], rows1, sem1).wait()
        pltpu.sync_copy(rows1, acc.at[dstseg.at[q0, k0 + 1]], add=True)
        return carry

    lax.fori_loop(0, lax.div(nch, 2), pair, 0)
    plsc.subcore_barrier()
    pltpu.sync_copy(
        acc.at[pl.ds(pl.multiple_of(s * ROWS_PER_TILE, 8), ROWS_PER_TILE)],
        out_hbm.at[c, pl.ds(pl.multiple_of(s * ROWS_PER_TILE, 8), ROWS_PER_TILE)],
    )


def _tc_prep(deg_parts, x):
    def body(deg_ref, x_ref, dis_ref, h1_ref):
        deg = deg_ref[0] + deg_ref[1]
        dis = lax.rsqrt(jnp.clip(deg, 1.0, None))
        dis_ref[...] = dis
        h1_ref[...] = x_ref[...] * dis

    return pl.pallas_call(
        body,
        grid=(NB,),
        in_specs=[
            pl.BlockSpec((NCORES, BR, D), lambda r: (0, r, 0)),
            pl.BlockSpec((BR, D), lambda r: (r, 0)),
        ],
        out_specs=[pl.BlockSpec((BR, D), lambda r: (r, 0))] * 2,
        out_shape=[jax.ShapeDtypeStruct((N_PAD, D), jnp.float32)] * 2,
    )(deg_parts, x)


def _tc_layer(parts, dis_m, W, b):
    def body(p_ref, dis_ref, w_ref, b_ref, o_ref):
        dis = dis_ref[...]
        z = (p_ref[0] + p_ref[1]) * dis
        h = jnp.dot(z, w_ref[...], preferred_element_type=jnp.float32) + b_ref[...]
        o_ref[...] = jnp.maximum(h, 0.0) * dis

    return pl.pallas_call(
        body,
        grid=(NB,),
        in_specs=[
            pl.BlockSpec((NCORES, BR, D), lambda r: (0, r, 0)),
            pl.BlockSpec((BR, D), lambda r: (r, 0)),
            pl.BlockSpec((D, D), lambda r: (0, 0)),
            pl.BlockSpec((1, D), lambda r: (0, 0)),
        ],
        out_specs=pl.BlockSpec((BR, D), lambda r: (r, 0)),
        out_shape=jax.ShapeDtypeStruct((N_PAD, D), jnp.float32),
    )(parts, dis_m, W, b)


def _tc_final(parts, dis_m, batch_col, Wg2, bg2, si, Ws, bs,
              W1s, W1g, b1, W2, b2):
    def body(p_ref, dis_ref, bt_ref, wg_ref, bg_ref, si_ref, ws_ref, bs_ref,
             w1s_ref, w1g_ref, b1_ref, w2_ref, b2_ref, out_ref,
             sums_acc, cnt_acc):
        r = pl.program_id(0)

        @pl.when(r == 0)
        def _():
            sums_acc[...] = jnp.zeros((B, D), jnp.float32)
            cnt_acc[...] = jnp.zeros((B, D), jnp.float32)

        z = (p_ref[0] + p_ref[1]) * dis_ref[...]
        post = jnp.dot(z, wg_ref[...], preferred_element_type=jnp.float32)
        post = post + bg_ref[...]
        gids = lax.broadcasted_iota(jnp.int32, (BR, B), 1)
        oh = (bt_ref[...] == gids).astype(jnp.float32)
        dn = (((0,), (0,)), ((), ()))
        sums_acc[...] += lax.dot_general(
            oh, post, dn, preferred_element_type=jnp.float32)
        cnt_acc[...] += lax.dot_general(
            oh, jnp.ones((BR, D), jnp.float32), dn,
            preferred_element_type=jnp.float32)

        @pl.when(r == NB - 1)
        def _():
            ivdet = sums_acc[...] / jnp.clip(cnt_acc[...], 1.0, None)
            sys = jnp.dot(si_ref[...], ws_ref[...],
                          preferred_element_type=jnp.float32) + bs_ref[...]
            sys = jnp.maximum(sys, 0.0)
            hh = (jnp.dot(sys, w1s_ref[...], preferred_element_type=jnp.float32)
                  + jnp.dot(ivdet, w1g_ref[...],
                            preferred_element_type=jnp.float32)
                  + b1_ref[...])
            hh = jnp.maximum(hh, 0.0)
            out_ref[...] = jnp.dot(
                hh, w2_ref[...], preferred_element_type=jnp.float32) + b2_ref[...]

    full = lambda shape: pl.BlockSpec(shape, lambda r: tuple(0 for _ in shape))
    return pl.pallas_call(
        body,
        grid=(NB,),
        in_specs=[
            pl.BlockSpec((NCORES, BR, D), lambda r: (0, r, 0)),
            pl.BlockSpec((BR, D), lambda r: (r, 0)),
            pl.BlockSpec((BR, 1), lambda r: (r, 0)),
            full((D, D)),
            full((1, D)),
            full((B, DSYS)),
            full((DSYS, SYS_OUT)),
            full((1, SYS_OUT)),
            full((SYS_OUT, 128)),
            full((D, 128)),
            full((1, 128)),
            full((128, 2)),
            full((1, 2)),
        ],
        out_specs=pl.BlockSpec((B, 2), lambda r: (0, 0)),
        out_shape=jax.ShapeDtypeStruct((B, 2), jnp.float32),
        scratch_shapes=[
            pltpu.VMEM((B, D), jnp.float32),
            pltpu.VMEM((B, D), jnp.float32),
        ],
    )(parts, dis_m, batch_col, Wg2, bg2, si, Ws, bs, W1s, W1g, b1, W2, b2)


def kernel(sysevr_input, x, edge_index, batch, Ws, bs,
           Wg0, bg0, Wg1, bg1, Wg2, bg2, W1, b1, W2, b2):
    src_ch = edge_index[0].astype(jnp.int32).reshape(NCHT, CHUNK)
    dst_ch = edge_index[1].astype(jnp.int32).reshape(NCHT, CHUNK)
    x_pad = jnp.pad(x, ((0, N_PAD - N), (0, 0)))
    zeros_rows = jnp.zeros((ROWS_PER_TILE, D), jnp.float32)
    ones_rows = jnp.ones((CHUNK, D), jnp.float32)
    batch_col = jnp.concatenate(
        [batch.astype(jnp.int32), jnp.full((N_PAD - N,), B + 7, jnp.int32)]
    ).reshape(N_PAD, 1)

    deg_parts = _sc_deg(dst_ch, ones_rows, zeros_rows)
    dis_m, h = _tc_prep(deg_parts, x_pad)
    parts = _sc_conv(h, src_ch, dst_ch, zeros_rows)
    h = _tc_layer(parts, dis_m, Wg0, bg0.reshape(1, D))
    parts = _sc_conv(h, src_ch, dst_ch, zeros_rows)
    h = _tc_layer(parts, dis_m, Wg1, bg1.reshape(1, D))
    parts = _sc_conv(h, src_ch, dst_ch, zeros_rows)
    out = _tc_final(parts, dis_m, batch_col, Wg2, bg2.reshape(1, D),
                    sysevr_input, Ws, bs.reshape(1, SYS_OUT),
                    W1[:SYS_OUT], W1[SYS_OUT:], b1.reshape(1, 128),
                    W2, b2.reshape(1, 2))
    return out


# speedup vs baseline: 16.1042x; 1.2383x over previous
import functools

import jax
import jax.numpy as jnp
from jax import lax
from jax.experimental import pallas as pl
from jax.experimental.pallas import tpu as pltpu
from jax.experimental.pallas import tpu_sc as plsc

N = 10000
D = 128
B = 32
DSYS = 100
SYS_OUT = 512

NCORES = 2
NSUB = 16
CHUNK = 32
SUP = 8
NSUP = 1250
DW = 128
RPT = 624
BR = 2000
NB = N // BR

_mesh = plsc.VectorSubcoreMesh(
    core_axis_name="c", subcore_axis_name="s",
    num_cores=NCORES, num_subcores=NSUB,
)


def _conv_partition(c, s):
    nsup = jnp.where(s < 15, 39, 40)
    sbase = c * 625 + jnp.where(s < 15, s * 39, 585)
    return nsup, sbase


def _deg_partition(wid):
    nsup = jnp.where(wid < 30, 39, 40)
    sbase = jnp.where(wid < 30, wid * 39, 1170 + (wid - 30) * 40)
    return nsup, sbase


def _zero_rows(zeros_hbm, acc, s):
    pltpu.sync_copy(zeros_hbm, acc.at[pl.ds(pl.multiple_of(s * RPT, 8), RPT)])

    @pl.when(s == 15)
    def _():
        pltpu.sync_copy(zeros_hbm.at[pl.ds(0, 16)], acc.at[pl.ds(9984, 16)])


def _copy_out(acc, out_hbm, c, s):
    pltpu.sync_copy(
        acc.at[pl.ds(pl.multiple_of(s * RPT, 8), RPT)],
        out_hbm.at[c, pl.ds(pl.multiple_of(s * RPT, 8), RPT)],
    )

    @pl.when(s == 15)
    def _():
        pltpu.sync_copy(acc.at[pl.ds(9984, 16)], out_hbm.at[c, pl.ds(9984, 16)])


@functools.partial(
    pl.kernel,
    out_type=jax.ShapeDtypeStruct((NCORES, N, DW), jnp.float32),
    mesh=_mesh,
    scratch_types=[
        pltpu.VMEM((SUP, CHUNK), jnp.int32),
        pltpu.VMEM((CHUNK, DW), jnp.float32),
        pltpu.VMEM_SHARED((N, DW), jnp.float32),
    ],
)
def _sc_deg(edges_hbm, ones_hbm, zeros_hbm, out_hbm, seg_v, ones_v, acc):
    c = lax.axis_index("c")
    s = lax.axis_index("s")
    wid = c * NSUB + s
    nsup, sbase = _deg_partition(wid)
    pltpu.sync_copy(ones_hbm, ones_v)
    _zero_rows(zeros_hbm, acc, s)
    plsc.subcore_barrier()

    def sup_step(g, carry):
        pltpu.sync_copy(edges_hbm.at[1, sbase + g], seg_v)

        def step(k, carry2):
            pltpu.sync_copy(ones_v, acc.at[seg_v.at[k]], add=True)
            return carry2

        lax.fori_loop(0, SUP, step, 0)
        return carry

    lax.fori_loop(0, nsup, sup_step, 0)
    plsc.subcore_barrier()
    _copy_out(acc, out_hbm, c, s)


@functools.partial(
    pl.kernel,
    out_type=jax.ShapeDtypeStruct((NCORES, N, D), jnp.float32),
    mesh=_mesh,
    scratch_types=[
        pltpu.VMEM((2, SUP, CHUNK), jnp.int32),
        pltpu.VMEM((2, SUP, CHUNK), jnp.int32),
        [pltpu.VMEM((CHUNK, D), jnp.float32)] * 4,
        [pltpu.SemaphoreType.DMA] * 4,
        pltpu.VMEM_SHARED((N, D), jnp.float32),
    ],
)
def _sc_conv(h_hbm, edges_hbm, zeros_hbm, out_hbm,
             srcseg, dstseg, rows, sems, acc):
    c = lax.axis_index("c")
    s = lax.axis_index("s")
    nsup, sbase = _conv_partition(c, s)
    _zero_rows(zeros_hbm, acc, s)
    plsc.subcore_barrier()
    pltpu.sync_copy(edges_hbm.at[0, sbase], srcseg.at[0])
    pltpu.sync_copy(edges_hbm.at[1, sbase], dstseg.at[0])
    for k in range(4):
        pltpu.async_copy(h_hbm.at[srcseg.at[0, k]], rows[k], sems[k])

    def sup_step(g, carry):
        p = lax.rem(g, 2)
        q = 1 - p
        nxt = sbase + g + 1
        have_next = g + 1 < nsup
        for k in range(4):
            pltpu.make_async_copy(
                h_hbm.at[srcseg.at[p, k]], rows[k], sems[k]).wait()
            pltpu.sync_copy(rows[k], acc.at[dstseg.at[p, k]], add=True)
            pltpu.async_copy(h_hbm.at[srcseg.at[p, 4 + k]], rows[k], sems[k])

        @pl.when(have_next)
        def _():
            pltpu.sync_copy(edges_hbm.at[0, nxt], srcseg.at[q])
            pltpu.sync_copy(edges_hbm.at[1, nxt], dstseg.at[q])

        for k in range(4):
            pltpu.make_async_copy(
                h_hbm.at[srcseg.at[p, 4 + k]], rows[k], sems[k]).wait()
            pltpu.sync_copy(rows[k], acc.at[dstseg.at[p, 4 + k]], add=True)

            @pl.when(have_next)
            def _():
                pltpu.async_copy(h_hbm.at[srcseg.at[q, k]], rows[k], sems[k])

        return carry

    lax.fori_loop(0, nsup, sup_step, 0)
    plsc.subcore_barrier()
    _copy_out(acc, out_hbm, c, s)


def _tc_prep(deg_parts, x):
    def body(deg_ref, x_ref, dis_ref, h1_ref):
        deg = deg_ref[0, :, 0:1] + deg_ref[1, :, 0:1]
        dis = lax.rsqrt(jnp.clip(deg, 1.0, None))
        dis = jnp.broadcast_to(dis, (BR, D))
        dis_ref[...] = dis
        h1_ref[...] = x_ref[...] * dis

    return pl.pallas_call(
        body,
        grid=(NB,),
        in_specs=[
            pl.BlockSpec((NCORES, BR, DW), lambda r: (0, r, 0)),
            pl.BlockSpec((BR, D), lambda r: (r, 0)),
        ],
        out_specs=[pl.BlockSpec((BR, D), lambda r: (r, 0))] * 2,
        out_shape=[jax.ShapeDtypeStruct((N, D), jnp.float32)] * 2,
    )(deg_parts, x)


def _tc_layer(parts, dis_m, W, b):
    def body(p_ref, dis_ref, w_ref, b_ref, o_ref):
        dis = dis_ref[...]
        z = (p_ref[0] + p_ref[1]) * dis
        h = jnp.dot(z, w_ref[...], preferred_element_type=jnp.float32) + b_ref[...]
        o_ref[...] = jnp.maximum(h, 0.0) * dis

    return pl.pallas_call(
        body,
        grid=(NB,),
        in_specs=[
            pl.BlockSpec((NCORES, BR, D), lambda r: (0, r, 0)),
            pl.BlockSpec((BR, D), lambda r: (r, 0)),
            pl.BlockSpec((D, D), lambda r: (0, 0)),
            pl.BlockSpec((1, D), lambda r: (0, 0)),
        ],
        out_specs=pl.BlockSpec((BR, D), lambda r: (r, 0)),
        out_shape=jax.ShapeDtypeStruct((N, D), jnp.float32),
    )(parts, dis_m, W, b)


def _tc_final(parts, dis_m, batch_col, Wg2, bg2, si, Ws, bs,
              W1s, W1g, b1, W2, b2):
    def body(p_ref, dis_ref, bt_ref, wg_ref, bg_ref, si_ref, ws_ref, bs_ref,
             w1s_ref, w1g_ref, b1_ref, w2_ref, b2_ref, out_ref,
             sums_acc, cnt_acc):
        r = pl.program_id(0)

        @pl.when(r == 0)
        def _():
            sums_acc[...] = jnp.zeros((B, D), jnp.float32)
            cnt_acc[...] = jnp.zeros((B, D), jnp.float32)

        z = (p_ref[0] + p_ref[1]) * dis_ref[...]
        post = jnp.dot(z, wg_ref[...], preferred_element_type=jnp.float32)
        post = post + bg_ref[...]
        gids = lax.broadcasted_iota(jnp.int32, (BR, B), 1)
        oh = (bt_ref[...] == gids).astype(jnp.float32)
        dn = (((0,), (0,)), ((), ()))
        sums_acc[...] += lax.dot_general(
            oh, post, dn, preferred_element_type=jnp.float32)
        cnt_acc[...] += lax.dot_general(
            oh, jnp.ones((BR, D), jnp.float32), dn,
            preferred_element_type=jnp.float32)

        @pl.when(r == NB - 1)
        def _():
            ivdet = sums_acc[...] / jnp.clip(cnt_acc[...], 1.0, None)
            sys = jnp.dot(si_ref[...], ws_ref[...],
                          preferred_element_type=jnp.float32) + bs_ref[...]
            sys = jnp.maximum(sys, 0.0)
            hh = (jnp.dot(sys, w1s_ref[...], preferred_element_type=jnp.float32)
                  + jnp.dot(ivdet, w1g_ref[...],
                            preferred_element_type=jnp.float32)
                  + b1_ref[...])
            hh = jnp.maximum(hh, 0.0)
            out_ref[...] = jnp.dot(
                hh, w2_ref[...], preferred_element_type=jnp.float32) + b2_ref[...]

    full = lambda shape: pl.BlockSpec(shape, lambda r: tuple(0 for _ in shape))
    return pl.pallas_call(
        body,
        grid=(NB,),
        in_specs=[
            pl.BlockSpec((NCORES, BR, D), lambda r: (0, r, 0)),
            pl.BlockSpec((BR, D), lambda r: (r, 0)),
            pl.BlockSpec((BR, 1), lambda r: (r, 0)),
            full((D, D)),
            full((1, D)),
            full((B, DSYS)),
            full((DSYS, SYS_OUT)),
            full((1, SYS_OUT)),
            full((SYS_OUT, 128)),
            full((D, 128)),
            full((1, 128)),
            full((128, 2)),
            full((1, 2)),
        ],
        out_specs=pl.BlockSpec((B, 2), lambda r: (0, 0)),
        out_shape=jax.ShapeDtypeStruct((B, 2), jnp.float32),
        scratch_shapes=[
            pltpu.VMEM((B, D), jnp.float32),
            pltpu.VMEM((B, D), jnp.float32),
        ],
    )(parts, dis_m, batch_col, Wg2, bg2, si, Ws, bs, W1s, W1g, b1, W2, b2)


def kernel(sysevr_input, x, edge_index, batch, Ws, bs,
           Wg0, bg0, Wg1, bg1, Wg2, bg2, W1, b1, W2, b2):
    edges_sup = edge_index.astype(jnp.int32).reshape(2, NSUP, SUP, CHUNK)
    zeros_rows = jnp.zeros((RPT, D), jnp.float32)
    zeros_deg = jnp.zeros((RPT, DW), jnp.float32)
    ones_deg = jnp.ones((CHUNK, DW), jnp.float32)
    batch_col = batch.astype(jnp.int32).reshape(N, 1)

    deg_parts = _sc_deg(edges_sup, ones_deg, zeros_deg)
    dis_m, h = _tc_prep(deg_parts, x)
    parts = _sc_conv(h, edges_sup, zeros_rows)
    h = _tc_layer(parts, dis_m, Wg0, bg0.reshape(1, D))
    parts = _sc_conv(h, edges_sup, zeros_rows)
    h = _tc_layer(parts, dis_m, Wg1, bg1.reshape(1, D))
    parts = _sc_conv(h, edges_sup, zeros_rows)
    out = _tc_final(parts, dis_m, batch_col, Wg2, bg2.reshape(1, D),
                    sysevr_input, Ws, bs.reshape(1, SYS_OUT),
                    W1[:SYS_OUT], W1[SYS_OUT:], b1.reshape(1, 128),
                    W2, b2.reshape(1, 2))
    return out
